# baseline (speedup 1.0000x reference)
import functools

import jax
import jax.numpy as jnp
from jax import lax
from jax.experimental import pallas as pl
from jax.experimental.pallas import tpu as pltpu
from jax.experimental.pallas import tpu_sc as plsc

B = 32
N = 2048
K = 32
EPS = 1e-5
NW = 32


def _init_p1(xT_ref, w_ref, h_ref, st_ref):
    h = jnp.dot(xT_ref[0], w_ref[...], preferred_element_type=jnp.float32)

    @pl.when(pl.program_id(0) == 0)
    def _():
        st_ref[...] = jnp.zeros_like(st_ref)

    st_ref[0:1, :] += jnp.sum(h, axis=0, keepdims=True)
    st_ref[1:2, :] += jnp.sum(h * h, axis=0, keepdims=True)
    h_ref[0] = h


def _init_p2(h_ref, sc_ref, sh_ref, w_ref, h2_ref, st_ref):
    f = jnp.maximum(h_ref[0] * sc_ref[...] + sh_ref[...], 0.0)
    h2 = jnp.dot(f, w_ref[...], preferred_element_type=jnp.float32)

    @pl.when(pl.program_id(0) == 0)
    def _():
        st_ref[...] = jnp.zeros_like(st_ref)

    st_ref[0:1, :] += jnp.sum(h2, axis=0, keepdims=True)
    st_ref[1:2, :] += jnp.sum(h2 * h2, axis=0, keepdims=True)
    h2_ref[0] = h2


def _init_p3(h_ref, sc_ref, sh_ref, f_ref):
    f_ref[0] = jnp.maximum(h_ref[0] * sc_ref[...] + sh_ref[...], 0.0)


def _fps_kernel(S, Nn, x_ref, fps_ref, nxz_ref):
    iota_n = lax.broadcasted_iota(jnp.int32, (B, Nn), 1)
    iota_s = lax.broadcasted_iota(jnp.int32, (B, S), 1)
    iota_z = lax.broadcasted_iota(jnp.int32, (B, 8, S), 2)
    boff = lax.broadcasted_iota(jnp.int32, (B, 1), 0) * Nn
    x0 = x_ref[:, 0, :]
    x1 = x_ref[:, 1, :]
    x2 = x_ref[:, 2, :]

    def body(i, carry):
        dist, far, fps, nxz = carry
        sel = iota_n == far
        c0 = jnp.sum(jnp.where(sel, x0, 0.0), axis=1, keepdims=True)
        c1 = jnp.sum(jnp.where(sel, x1, 0.0), axis=1, keepdims=True)
        c2 = jnp.sum(jnp.where(sel, x2, 0.0), axis=1, keepdims=True)
        crow = jnp.concatenate(
            [c0, c1, c2, jnp.zeros((B, 5), jnp.float32)], axis=1)
        fps = jnp.where(iota_s == i, far + boff, fps)
        nxz = jnp.where(iota_z == i, crow[:, :, None], nxz)
        d = (x0 - c0) ** 2 + (x1 - c1) ** 2 + (x2 - c2) ** 2
        dist = jnp.minimum(dist, d)
        m = jnp.max(dist, axis=1, keepdims=True)
        far = jnp.min(jnp.where(dist == m, iota_n, Nn), axis=1, keepdims=True)
        return dist, far.astype(jnp.int32), fps, nxz

    init = (jnp.full((B, Nn), 1e10, jnp.float32),
            jnp.zeros((B, 1), jnp.int32),
            jnp.zeros((B, S), jnp.int32),
            jnp.zeros((B, 8, S), jnp.float32))
    _, _, fps, nxz = lax.fori_loop(0, S, body, init)
    fps_ref[...] = fps
    nxz_ref[...] = nxz


def _fps(x, S, Nn):
    fps, nxz = pl.pallas_call(
        functools.partial(_fps_kernel, S, Nn),
        out_shape=(jax.ShapeDtypeStruct((B, S), jnp.int32),
                   jax.ShapeDtypeStruct((B, 8, S), jnp.float32)),
    )(x)
    nxT = jnp.transpose(nxz[:, :3, :], (0, 2, 1))
    return fps, nxz, nxT


def _sc_topk_call(S, Nn, sqdT):
    SB = S // 16
    gpw = (B * SB) // NW
    C1 = Nn // 16
    C2 = C1 // 16
    mesh = plsc.VectorSubcoreMesh(core_axis_name="c", subcore_axis_name="s")

    GSZ = Nn * 16
    OSZ = K * 16

    @functools.partial(
        pl.kernel, mesh=mesh,
        out_type=jax.ShapeDtypeStruct((B * SB * OSZ,), jnp.int32),
        compiler_params=pltpu.CompilerParams(needs_layout_passes=False),
        scratch_types=[pltpu.VMEM((2 * GSZ,), jnp.float32),
                       pltpu.VMEM((C1 * 16,), jnp.float32),
                       pltpu.VMEM((OSZ,), jnp.int32),
                       pltpu.SemaphoreType.DMA,
                       pltpu.SemaphoreType.DMA],
    )
    def k(sqdT_hbm, out_hbm, d_v, m1_v, o_v, sem0, sem1):
        wid = lax.axis_index("s") * 2 + lax.axis_index("c")
        lane = lax.broadcasted_iota(jnp.int32, (16,), 0)
        inf16 = jnp.full((16,), jnp.inf, jnp.float32)
        sems = [sem0, sem1]
        g0 = wid * gpw
        for par in range(2):
            pltpu.async_copy(sqdT_hbm.at[pl.ds((g0 + par) * GSZ, GSZ)],
                             d_v.at[pl.ds(par * GSZ, GSZ)], sems[par])

        def group_body(par, gl):
            g = g0 + gl
            b = g // SB
            off = par * GSZ
            pltpu.make_async_copy(sqdT_hbm.at[pl.ds(0, GSZ)],
                                  d_v.at[pl.ds(off, GSZ)], sems[par]).wait()

            def dval(pvec):
                return plsc.load_gather(d_v, [off + pvec * 16 + lane])

            def build1(c, _c):
                acc = dval(c * 16)
                for j in range(1, 16):
                    acc = jnp.minimum(acc, dval(c * 16 + j))
                plsc.store_scatter(m1_v, [c * 16 + lane], acc)
                return 0

            lax.fori_loop(0, C1, build1, 0)
            m2 = []
            for c2 in range(C2):
                acc = m1_v[pl.ds(c2 * 256, 16)]
                for t in range(1, 16):
                    acc = jnp.minimum(acc, m1_v[pl.ds((c2 * 16 + t) * 16, 16)])
                m2.append(acc)

            def extract(j, m2c):
                gm, sv = inf16, jnp.zeros((16,), jnp.int32)
                for c2 in range(C2):
                    lt = m2c[c2] < gm
                    gm = jnp.where(lt, m2c[c2], gm)
                    sv = jnp.where(lt, c2, sv)
                cm, cv = inf16, jnp.zeros((16,), jnp.int32)
                for t in range(16):
                    cc = sv * 16 + t
                    v = plsc.load_gather(m1_v, [cc * 16 + lane])
                    lt = v < cm
                    cm = jnp.where(lt, v, cm)
                    cv = jnp.where(lt, cc, cv)
                pm, pv = inf16, jnp.zeros((16,), jnp.int32)
                for u in range(16):
                    pp = cv * 16 + u
                    v = dval(pp)
                    lt = v < pm
                    pm = jnp.where(lt, v, pm)
                    pv = jnp.where(lt, pp, pv)
                plsc.store_scatter(o_v, [j * 16 + lane], pv + b * Nn)
                plsc.store_scatter(d_v, [off + pv * 16 + lane], inf16)
                nm = inf16
                for u in range(16):
                    nm = jnp.minimum(nm, dval(cv * 16 + u))
                plsc.store_scatter(m1_v, [cv * 16 + lane], nm)
                ns = inf16
                for t in range(16):
                    ns = jnp.minimum(
                        ns, plsc.load_gather(m1_v, [(sv * 16 + t) * 16 + lane]))
                return tuple(jnp.where(sv == c2, ns, m2c[c2])
                             for c2 in range(C2))

            lax.fori_loop(0, K, extract, tuple(m2))
            pltpu.sync_copy(o_v, out_hbm.at[pl.ds(g * OSZ, OSZ)])

            @pl.when(gl + 2 < gpw)
            def _():
                pltpu.async_copy(sqdT_hbm.at[pl.ds((g + 2) * GSZ, GSZ)],
                                 d_v.at[pl.ds(off, GSZ)], sems[par])

        def pair_body(gp, _p):
            for par in range(2):
                group_body(par, gp * 2 + par)
            return 0

        lax.fori_loop(0, gpw // 2, pair_body, 0)

    return k(sqdT)


def _knn(nxT, coords, S, Nn):
    nx = nxT
    sqd = (jnp.sum(nx ** 2, -1)[:, :, None]
           + jnp.sum(coords ** 2, -1)[:, None, :]
           - 2.0 * jnp.einsum('bsd,bnd->bsn', nx, coords))
    SB = S // 16
    sqdg = jnp.transpose(sqd.reshape(B, SB, 16, Nn),
                         (0, 1, 3, 2)).reshape(-1)
    knng = _sc_topk_call(S, Nn, sqdg).reshape(B, SB, K, 16)
    return jnp.transpose(knng, (0, 1, 3, 2)).reshape(B, S, K)


_CH = 128
_NBUF = 2


def _sc_gather_call(V, D, M, table, gidx):
    rpw = M // NW
    nch = rpw // _CH
    mesh = plsc.VectorSubcoreMesh(core_axis_name="c", subcore_axis_name="s")

    @functools.partial(
        pl.kernel, mesh=mesh,
        out_type=jax.ShapeDtypeStruct((M, D), jnp.float32),
        scratch_types=[pltpu.VMEM((rpw,), jnp.int32),
                       pltpu.VMEM((_NBUF, _CH, D), jnp.float32),
                       pltpu.SemaphoreType.DMA,
                       pltpu.SemaphoreType.DMA],
    )
    def k(table_hbm, idx_hbm, out_hbm, idx_v, rows_v, sem0, sem1):
        wid = lax.axis_index("s") * 2 + lax.axis_index("c")
        base = wid * rpw
        sems = [sem0, sem1]
        pltpu.sync_copy(idx_hbm.at[pl.ds(base, rpw)], idx_v)
        for b in range(min(_NBUF, nch)):
            pltpu.async_copy(table_hbm.at[idx_v.at[pl.ds(b * _CH, _CH)]],
                             rows_v.at[b], sems[b])

        def body(cg, _):
            for b in range(_NBUF):
                ci = cg * _NBUF + b
                pltpu.make_async_copy(table_hbm.at[pl.ds(0, _CH)],
                                      rows_v.at[b], sems[b]).wait()
                pltpu.sync_copy(rows_v.at[b],
                                out_hbm.at[pl.ds(base + ci * _CH, _CH)])
                nxt = ci + _NBUF

                @pl.when(nxt < nch)
                def _():
                    pltpu.async_copy(
                        table_hbm.at[idx_v.at[pl.ds(nxt * _CH, _CH)]],
                        rows_v.at[b], sems[b])
            return 0

        if nch <= _NBUF:
            for b in range(nch):
                pltpu.make_async_copy(table_hbm.at[pl.ds(0, _CH)],
                                      rows_v.at[b], sems[b]).wait()
                pltpu.sync_copy(rows_v.at[b],
                                out_hbm.at[pl.ds(base + b * _CH, _CH)])
        else:
            lax.fori_loop(0, nch // _NBUF, body, 0)

    return k(table, gidx)


def _gather_rows(table, gidx):
    V, D = table.shape
    (M,) = gidx.shape
    return _sc_gather_call(V, D, M, table, gidx)


def _xform_kernel(f_ref, wa_ref, wd_ref, u_ref, v_ref):
    f = f_ref[...]
    u_ref[...] = jnp.dot(f, wa_ref[...], preferred_element_type=jnp.float32)
    v_ref[...] = jnp.dot(f, wd_ref[...], preferred_element_type=jnp.float32)


def _xform(feats, W1):
    R, D = feats.shape
    O = W1.shape[0]
    wa = jnp.transpose(W1[:, :D])
    wd = jnp.transpose(W1[:, D:] - W1[:, :D])
    T = 4096
    return pl.pallas_call(
        _xform_kernel,
        grid=(R // T,),
        in_specs=[pl.BlockSpec((T, D), lambda i: (i, 0)),
                  pl.BlockSpec((D, O), lambda i: (0, 0)),
                  pl.BlockSpec((D, O), lambda i: (0, 0))],
        out_specs=(pl.BlockSpec((T, O), lambda i: (i, 0)),
                   pl.BlockSpec((T, O), lambda i: (i, 0))),
        out_shape=(jax.ShapeDtypeStruct((R, O), jnp.float32),
                   jax.ShapeDtypeStruct((R, O), jnp.float32)),
    )(feats, wa, wd)


def _grp_p1(G, u_ref, cv_ref, st_ref):
    u = u_ref[...]
    T, O = u.shape
    dvec = cv_ref[...]
    gsum = jnp.sum(u.reshape(G, K, O), axis=1)

    @pl.when(pl.program_id(0) == 0)
    def _():
        st_ref[...] = jnp.zeros_like(st_ref)

    st_ref[0:1, :] += (jnp.sum(u, axis=0, keepdims=True)
                       + K * jnp.sum(dvec, axis=0, keepdims=True))
    st_ref[1:2, :] += (jnp.sum(u * u, axis=0, keepdims=True)
                       + 2.0 * jnp.sum(gsum * dvec, axis=0, keepdims=True)
                       + K * jnp.sum(dvec * dvec, axis=0, keepdims=True))


def _grp_p2(G, u_ref, cv_ref, sc_ref, sh_ref, w_ref, h2_ref, st_ref):
    T, O = u_ref.shape
    dexp = jnp.broadcast_to(cv_ref[...][:, None, :], (G, K, O)).reshape(T, O)
    h = u_ref[...] + dexp
    f = jnp.maximum(h * sc_ref[...] + sh_ref[...], 0.0)
    h2 = jnp.dot(f, w_ref[...], preferred_element_type=jnp.float32)

    @pl.when(pl.program_id(0) == 0)
    def _():
        st_ref[...] = jnp.zeros_like(st_ref)

    st_ref[0:1, :] += jnp.sum(h2, axis=0, keepdims=True)
    st_ref[1:2, :] += jnp.sum(h2 * h2, axis=0, keepdims=True)
    h2_ref[...] = h2


def _grp_p3(G, h_ref, sc_ref, sh_ref, out_ref):
    T, O = h_ref.shape
    v = jnp.maximum(h_ref[...] * sc_ref[...] + sh_ref[...], 0.0)
    out_ref[...] = jnp.max(v.reshape(G, K, O), axis=1)


def _affine(st, cnt, gamma, beta):
    m = st[0] / cnt
    v = jnp.maximum(st[1] / cnt - m * m, 0.0)
    sc = gamma / jnp.sqrt(v + EPS)
    sh = beta - m * sc
    return sc.reshape(1, -1), sh.reshape(1, -1)


def _group_stage(u, cv, W2, g1, b1, g2, b2):
    R, O = u.shape
    T = 2048
    G = T // K
    grid = R // T
    w2T = jnp.transpose(W2)

    st1 = pl.pallas_call(
        functools.partial(_grp_p1, G),
        grid=(grid,),
        in_specs=[pl.BlockSpec((T, O), lambda i: (i, 0)),
                  pl.BlockSpec((G, O), lambda i: (i, 0))],
        out_specs=pl.BlockSpec((8, O), lambda i: (0, 0)),
        out_shape=jax.ShapeDtypeStruct((8, O), jnp.float32),
    )(u, cv)
    sc1, sh1 = _affine(st1, R, g1, b1)

    h2, st2 = pl.pallas_call(
        functools.partial(_grp_p2, G),
        grid=(grid,),
        in_specs=[pl.BlockSpec((T, O), lambda i: (i, 0)),
                  pl.BlockSpec((G, O), lambda i: (i, 0)),
                  pl.BlockSpec((1, O), lambda i: (0, 0)),
                  pl.BlockSpec((1, O), lambda i: (0, 0)),
                  pl.BlockSpec((O, O), lambda i: (0, 0))],
        out_specs=(pl.BlockSpec((T, O), lambda i: (i, 0)),
                   pl.BlockSpec((8, O), lambda i: (0, 0))),
        out_shape=(jax.ShapeDtypeStruct((R, O), jnp.float32),
                   jax.ShapeDtypeStruct((8, O), jnp.float32)),
    )(u, cv, sc1, sh1, w2T)
    sc2, sh2 = _affine(st2, R, g2, b2)

    out = pl.pallas_call(
        functools.partial(_grp_p3, G),
        grid=(grid,),
        in_specs=[pl.BlockSpec((T, O), lambda i: (i, 0)),
                  pl.BlockSpec((1, O), lambda i: (0, 0)),
                  pl.BlockSpec((1, O), lambda i: (0, 0))],
        out_specs=pl.BlockSpec((G, O), lambda i: (i, 0)),
        out_shape=jax.ShapeDtypeStruct((R // K, O), jnp.float32),
    )(h2, sc2, sh2)
    return out


def _tr_kernel(rows_ref, out_ref):
    S, O = rows_ref.shape[1], rows_ref.shape[2]
    ii = lax.broadcasted_iota(jnp.int32, (S, S), 0)
    jj = lax.broadcasted_iota(jnp.int32, (S, S), 1)
    eye = (ii == jj).astype(jnp.float32)
    out_ref[0] = lax.dot_general(rows_ref[0], eye, (((0,), (0,)), ((), ())),
                                 preferred_element_type=jnp.float32)


def kernel(x, w1, g1, be1, w2, g2, be2,
           s1w1, s1g1, s1be1, s1w2, s1g2, s1be2,
           s2w1, s2g1, s2be1, s2w2, s2g2, s2be2):
    S1, S2 = 512, 256

    xp8 = jnp.concatenate([x, jnp.zeros((B, 5, N), jnp.float32)], axis=1)
    xT = jnp.transpose(xp8, (0, 2, 1))
    w1Tp = jnp.concatenate(
        [jnp.transpose(w1), jnp.zeros((5, 64), jnp.float32)], axis=0)

    h1, st1 = pl.pallas_call(
        _init_p1,
        grid=(B,),
        in_specs=[pl.BlockSpec((1, N, 8), lambda i: (i, 0, 0)),
                  pl.BlockSpec((8, 64), lambda i: (0, 0))],
        out_specs=(pl.BlockSpec((1, N, 64), lambda i: (i, 0, 0)),
                   pl.BlockSpec((8, 64), lambda i: (0, 0))),
        out_shape=(jax.ShapeDtypeStruct((B, N, 64), jnp.float32),
                   jax.ShapeDtypeStruct((8, 64), jnp.float32)),
    )(xT, w1Tp)
    sc1, sh1 = _affine(st1, B * N, g1, be1)

    h2, st2 = pl.pallas_call(
        _init_p2,
        grid=(B,),
        in_specs=[pl.BlockSpec((1, N, 64), lambda i: (i, 0, 0)),
                  pl.BlockSpec((1, 64), lambda i: (0, 0)),
                  pl.BlockSpec((1, 64), lambda i: (0, 0)),
                  pl.BlockSpec((64, 64), lambda i: (0, 0))],
        out_specs=(pl.BlockSpec((1, N, 64), lambda i: (i, 0, 0)),
                   pl.BlockSpec((8, 64), lambda i: (0, 0))),
        out_shape=(jax.ShapeDtypeStruct((B, N, 64), jnp.float32),
                   jax.ShapeDtypeStruct((8, 64), jnp.float32)),
    )(h1, sc1, sh1, jnp.transpose(w2))
    sc2, sh2 = _affine(st2, B * N, g2, be2)

    feats = pl.pallas_call(
        _init_p3,
        grid=(B,),
        in_specs=[pl.BlockSpec((1, N, 64), lambda i: (i, 0, 0)),
                  pl.BlockSpec((1, 64), lambda i: (0, 0)),
                  pl.BlockSpec((1, 64), lambda i: (0, 0))],
        out_specs=pl.BlockSpec((1, N, 64), lambda i: (i, 0, 0)),
        out_shape=jax.ShapeDtypeStruct((B, N, 64), jnp.float32),
    )(h2, sc2, sh2)
    feats_flat = feats.reshape(B * N, 64)

    fps1, nxz1, nxT1 = _fps(x, S1, N)
    xyz = jnp.transpose(x[:, :3, :], (0, 2, 1))
    knn1 = _knn(nxT1, xyz, S1, N)

    u1t, v1t = _xform(feats_flat, s1w1)
    cv1 = _gather_rows(v1t, fps1.reshape(-1))
    u1 = _gather_rows(u1t, knn1.reshape(-1))
    feats1 = _group_stage(u1, cv1, s1w2, s1g1, s1be1,
                          s1g2, s1be2)

    fps2, _, nxT2 = _fps(nxz1, S2, S1)
    knn2 = _knn(nxT2, nxT1, S2, S1)

    u2t, v2t = _xform(feats1, s2w1)
    cv2 = _gather_rows(v2t, fps2.reshape(-1))
    u2 = _gather_rows(u2t, knn2.reshape(-1))
    feats2 = _group_stage(u2, cv2, s2w2, s2g1, s2be1,
                          s2g2, s2be2)

    out = pl.pallas_call(
        _tr_kernel,
        grid=(B,),
        in_specs=[pl.BlockSpec((1, S2, 256), lambda i: (i, 0, 0))],
        out_specs=pl.BlockSpec((1, 256, S2), lambda i: (i, 0, 0)),
        out_shape=jax.ShapeDtypeStruct((B, 256, S2), jnp.float32),
    )(feats2.reshape(B, S2, 256))
    return out


# speedup vs baseline: 1.0003x; 1.0003x over previous
import functools

import jax
import jax.numpy as jnp
from jax import lax
from jax.experimental import pallas as pl
from jax.experimental.pallas import tpu as pltpu
from jax.experimental.pallas import tpu_sc as plsc

B = 32
N = 2048
K = 32
EPS = 1e-5
NW = 32


def _init_p1(xT_ref, w_ref, h_ref, st_ref):
    h = jnp.dot(xT_ref[0], w_ref[...], preferred_element_type=jnp.float32)

    @pl.when(pl.program_id(0) == 0)
    def _():
        st_ref[...] = jnp.zeros_like(st_ref)

    st_ref[0:1, :] += jnp.sum(h, axis=0, keepdims=True)
    st_ref[1:2, :] += jnp.sum(h * h, axis=0, keepdims=True)
    h_ref[0] = h


def _init_p2(h_ref, sc_ref, sh_ref, w_ref, h2_ref, st_ref):
    f = jnp.maximum(h_ref[0] * sc_ref[...] + sh_ref[...], 0.0)
    h2 = jnp.dot(f, w_ref[...], preferred_element_type=jnp.float32)

    @pl.when(pl.program_id(0) == 0)
    def _():
        st_ref[...] = jnp.zeros_like(st_ref)

    st_ref[0:1, :] += jnp.sum(h2, axis=0, keepdims=True)
    st_ref[1:2, :] += jnp.sum(h2 * h2, axis=0, keepdims=True)
    h2_ref[0] = h2


def _init_p3(h_ref, sc_ref, sh_ref, f_ref):
    f_ref[0] = jnp.maximum(h_ref[0] * sc_ref[...] + sh_ref[...], 0.0)


def _fps_kernel(S, Nn, x_ref, fps_ref, nxz_ref):
    iota_n = lax.broadcasted_iota(jnp.int32, (B, Nn), 1)
    iota_s = lax.broadcasted_iota(jnp.int32, (B, S), 1)
    iota_z = lax.broadcasted_iota(jnp.int32, (B, 8, S), 2)
    boff = lax.broadcasted_iota(jnp.int32, (B, 1), 0) * Nn
    x0 = x_ref[:, 0, :]
    x1 = x_ref[:, 1, :]
    x2 = x_ref[:, 2, :]

    def body(i, carry):
        dist, far, fps, nxz = carry
        sel = iota_n == far
        c0 = jnp.sum(jnp.where(sel, x0, 0.0), axis=1, keepdims=True)
        c1 = jnp.sum(jnp.where(sel, x1, 0.0), axis=1, keepdims=True)
        c2 = jnp.sum(jnp.where(sel, x2, 0.0), axis=1, keepdims=True)
        crow = jnp.concatenate(
            [c0, c1, c2, jnp.zeros((B, 5), jnp.float32)], axis=1)
        fps = jnp.where(iota_s == i, far + boff, fps)
        nxz = jnp.where(iota_z == i, crow[:, :, None], nxz)
        d = (x0 - c0) ** 2 + (x1 - c1) ** 2 + (x2 - c2) ** 2
        dist = jnp.minimum(dist, d)
        m = jnp.max(dist, axis=1, keepdims=True)
        far = jnp.min(jnp.where(dist == m, iota_n, Nn), axis=1, keepdims=True)
        return dist, far.astype(jnp.int32), fps, nxz

    init = (jnp.full((B, Nn), 1e10, jnp.float32),
            jnp.zeros((B, 1), jnp.int32),
            jnp.zeros((B, S), jnp.int32),
            jnp.zeros((B, 8, S), jnp.float32))
    _, _, fps, nxz = lax.fori_loop(0, S, body, init)
    fps_ref[...] = fps
    nxz_ref[...] = nxz


def _fps(x, S, Nn):
    fps, nxz = pl.pallas_call(
        functools.partial(_fps_kernel, S, Nn),
        out_shape=(jax.ShapeDtypeStruct((B, S), jnp.int32),
                   jax.ShapeDtypeStruct((B, 8, S), jnp.float32)),
    )(x)
    nxT = jnp.transpose(nxz[:, :3, :], (0, 2, 1))
    return fps, nxz, nxT


def _sc_topk_call(S, Nn, sqdT):
    SB = S // 16
    gpw = (B * SB) // NW
    C1 = Nn // 16
    C2 = C1 // 16
    mesh = plsc.VectorSubcoreMesh(core_axis_name="c", subcore_axis_name="s")

    GSZ = Nn * 16
    OSZ = K * 16

    @functools.partial(
        pl.kernel, mesh=mesh,
        out_type=jax.ShapeDtypeStruct((B * SB * OSZ,), jnp.int32),
        compiler_params=pltpu.CompilerParams(needs_layout_passes=False),
        scratch_types=[pltpu.VMEM((2 * GSZ,), jnp.float32),
                       pltpu.VMEM((C1 * 16,), jnp.float32),
                       pltpu.VMEM((OSZ,), jnp.int32),
                       pltpu.SemaphoreType.DMA,
                       pltpu.SemaphoreType.DMA],
    )
    def k(sqdT_hbm, out_hbm, d_v, m1_v, o_v, sem0, sem1):
        wid = lax.axis_index("s") * 2 + lax.axis_index("c")
        lane = lax.broadcasted_iota(jnp.int32, (16,), 0)
        inf16 = jnp.full((16,), jnp.inf, jnp.float32)
        sems = [sem0, sem1]
        g0 = wid * gpw
        for par in range(2):
            pltpu.async_copy(sqdT_hbm.at[pl.ds((g0 + par) * GSZ, GSZ)],
                             d_v.at[pl.ds(par * GSZ, GSZ)], sems[par])

        def group_body(par, gl):
            g = g0 + gl
            b = g // SB
            off = par * GSZ
            pltpu.make_async_copy(sqdT_hbm.at[pl.ds(0, GSZ)],
                                  d_v.at[pl.ds(off, GSZ)], sems[par]).wait()

            def dval(pvec):
                return plsc.load_gather(d_v, [off + pvec * 16 + lane])

            def build1(c, _c):
                acc = dval(c * 16)
                for j in range(1, 16):
                    acc = jnp.minimum(acc, dval(c * 16 + j))
                plsc.store_scatter(m1_v, [c * 16 + lane], acc)
                return 0

            lax.fori_loop(0, C1, build1, 0)
            m2 = []
            for c2 in range(C2):
                acc = m1_v[pl.ds(c2 * 256, 16)]
                for t in range(1, 16):
                    acc = jnp.minimum(acc, m1_v[pl.ds((c2 * 16 + t) * 16, 16)])
                m2.append(acc)

            def extract(j, m2c):
                gm, sv = inf16, jnp.zeros((16,), jnp.int32)
                for c2 in range(C2):
                    lt = m2c[c2] < gm
                    gm = jnp.where(lt, m2c[c2], gm)
                    sv = jnp.where(lt, c2, sv)
                cm, cm2, cv = inf16, inf16, jnp.zeros((16,), jnp.int32)
                for t in range(16):
                    cc = sv * 16 + t
                    v = plsc.load_gather(m1_v, [cc * 16 + lane])
                    lt = v < cm
                    cm2 = jnp.where(lt, cm, jnp.minimum(v, cm2))
                    cm = jnp.where(lt, v, cm)
                    cv = jnp.where(lt, cc, cv)
                pm, pm2, pv = inf16, inf16, jnp.zeros((16,), jnp.int32)
                for u in range(16):
                    pp = cv * 16 + u
                    v = dval(pp)
                    lt = v < pm
                    pm2 = jnp.where(lt, pm, jnp.minimum(v, pm2))
                    pm = jnp.where(lt, v, pm)
                    pv = jnp.where(lt, pp, pv)
                plsc.store_scatter(o_v, [j * 16 + lane], pv + b * Nn)
                plsc.store_scatter(d_v, [off + pv * 16 + lane], inf16)
                plsc.store_scatter(m1_v, [cv * 16 + lane], pm2)
                ns = jnp.minimum(cm2, pm2)
                return tuple(jnp.where(sv == c2, ns, m2c[c2])
                             for c2 in range(C2))

            lax.fori_loop(0, K, extract, tuple(m2))
            pltpu.sync_copy(o_v, out_hbm.at[pl.ds(g * OSZ, OSZ)])

            @pl.when(gl + 2 < gpw)
            def _():
                pltpu.async_copy(sqdT_hbm.at[pl.ds((g + 2) * GSZ, GSZ)],
                                 d_v.at[pl.ds(off, GSZ)], sems[par])

        def pair_body(gp, _p):
            for par in range(2):
                group_body(par, gp * 2 + par)
            return 0

        lax.fori_loop(0, gpw // 2, pair_body, 0)

    return k(sqdT)


def _knn(nxT, coords, S, Nn):
    nx = nxT
    sqd = (jnp.sum(nx ** 2, -1)[:, :, None]
           + jnp.sum(coords ** 2, -1)[:, None, :]
           - 2.0 * jnp.einsum('bsd,bnd->bsn', nx, coords))
    SB = S // 16
    sqdg = jnp.transpose(sqd.reshape(B, SB, 16, Nn),
                         (0, 1, 3, 2)).reshape(-1)
    knng = _sc_topk_call(S, Nn, sqdg).reshape(B, SB, K, 16)
    return jnp.transpose(knng, (0, 1, 3, 2)).reshape(B, S, K)


_CH = 128
_NBUF = 2


def _sc_gather_call(V, D, M, table, gidx):
    rpw = M // NW
    nch = rpw // _CH
    mesh = plsc.VectorSubcoreMesh(core_axis_name="c", subcore_axis_name="s")

    @functools.partial(
        pl.kernel, mesh=mesh,
        out_type=jax.ShapeDtypeStruct((M, D), jnp.float32),
        scratch_types=[pltpu.VMEM((rpw,), jnp.int32),
                       pltpu.VMEM((_NBUF, _CH, D), jnp.float32),
                       pltpu.SemaphoreType.DMA,
                       pltpu.SemaphoreType.DMA],
    )
    def k(table_hbm, idx_hbm, out_hbm, idx_v, rows_v, sem0, sem1):
        wid = lax.axis_index("s") * 2 + lax.axis_index("c")
        base = wid * rpw
        sems = [sem0, sem1]
        pltpu.sync_copy(idx_hbm.at[pl.ds(base, rpw)], idx_v)
        for b in range(min(_NBUF, nch)):
            pltpu.async_copy(table_hbm.at[idx_v.at[pl.ds(b * _CH, _CH)]],
                             rows_v.at[b], sems[b])

        def body(cg, _):
            for b in range(_NBUF):
                ci = cg * _NBUF + b
                pltpu.make_async_copy(table_hbm.at[pl.ds(0, _CH)],
                                      rows_v.at[b], sems[b]).wait()
                pltpu.sync_copy(rows_v.at[b],
                                out_hbm.at[pl.ds(base + ci * _CH, _CH)])
                nxt = ci + _NBUF

                @pl.when(nxt < nch)
                def _():
                    pltpu.async_copy(
                        table_hbm.at[idx_v.at[pl.ds(nxt * _CH, _CH)]],
                        rows_v.at[b], sems[b])
            return 0

        if nch <= _NBUF:
            for b in range(nch):
                pltpu.make_async_copy(table_hbm.at[pl.ds(0, _CH)],
                                      rows_v.at[b], sems[b]).wait()
                pltpu.sync_copy(rows_v.at[b],
                                out_hbm.at[pl.ds(base + b * _CH, _CH)])
        else:
            lax.fori_loop(0, nch // _NBUF, body, 0)

    return k(table, gidx)


def _gather_rows(table, gidx):
    V, D = table.shape
    (M,) = gidx.shape
    return _sc_gather_call(V, D, M, table, gidx)


def _xform_kernel(f_ref, wa_ref, wd_ref, u_ref, v_ref):
    f = f_ref[...]
    u_ref[...] = jnp.dot(f, wa_ref[...], preferred_element_type=jnp.float32)
    v_ref[...] = jnp.dot(f, wd_ref[...], preferred_element_type=jnp.float32)


def _xform(feats, W1):
    R, D = feats.shape
    O = W1.shape[0]
    wa = jnp.transpose(W1[:, :D])
    wd = jnp.transpose(W1[:, D:] - W1[:, :D])
    T = 4096
    return pl.pallas_call(
        _xform_kernel,
        grid=(R // T,),
        in_specs=[pl.BlockSpec((T, D), lambda i: (i, 0)),
                  pl.BlockSpec((D, O), lambda i: (0, 0)),
                  pl.BlockSpec((D, O), lambda i: (0, 0))],
        out_specs=(pl.BlockSpec((T, O), lambda i: (i, 0)),
                   pl.BlockSpec((T, O), lambda i: (i, 0))),
        out_shape=(jax.ShapeDtypeStruct((R, O), jnp.float32),
                   jax.ShapeDtypeStruct((R, O), jnp.float32)),
    )(feats, wa, wd)


def _grp_p1(G, u_ref, cv_ref, st_ref):
    u = u_ref[...]
    T, O = u.shape
    dvec = cv_ref[...]
    gsum = jnp.sum(u.reshape(G, K, O), axis=1)

    @pl.when(pl.program_id(0) == 0)
    def _():
        st_ref[...] = jnp.zeros_like(st_ref)

    st_ref[0:1, :] += (jnp.sum(u, axis=0, keepdims=True)
                       + K * jnp.sum(dvec, axis=0, keepdims=True))
    st_ref[1:2, :] += (jnp.sum(u * u, axis=0, keepdims=True)
                       + 2.0 * jnp.sum(gsum * dvec, axis=0, keepdims=True)
                       + K * jnp.sum(dvec * dvec, axis=0, keepdims=True))


def _grp_p2(G, u_ref, cv_ref, sc_ref, sh_ref, w_ref, h2_ref, st_ref):
    T, O = u_ref.shape
    dexp = jnp.broadcast_to(cv_ref[...][:, None, :], (G, K, O)).reshape(T, O)
    h = u_ref[...] + dexp
    f = jnp.maximum(h * sc_ref[...] + sh_ref[...], 0.0)
    h2 = jnp.dot(f, w_ref[...], preferred_element_type=jnp.float32)

    @pl.when(pl.program_id(0) == 0)
    def _():
        st_ref[...] = jnp.zeros_like(st_ref)

    st_ref[0:1, :] += jnp.sum(h2, axis=0, keepdims=True)
    st_ref[1:2, :] += jnp.sum(h2 * h2, axis=0, keepdims=True)
    h2_ref[...] = h2


def _grp_p3(G, h_ref, sc_ref, sh_ref, out_ref):
    T, O = h_ref.shape
    v = jnp.maximum(h_ref[...] * sc_ref[...] + sh_ref[...], 0.0)
    out_ref[...] = jnp.max(v.reshape(G, K, O), axis=1)


def _affine(st, cnt, gamma, beta):
    m = st[0] / cnt
    v = jnp.maximum(st[1] / cnt - m * m, 0.0)
    sc = gamma / jnp.sqrt(v + EPS)
    sh = beta - m * sc
    return sc.reshape(1, -1), sh.reshape(1, -1)


def _group_stage(u, cv, W2, g1, b1, g2, b2):
    R, O = u.shape
    T = 2048
    G = T // K
    grid = R // T
    w2T = jnp.transpose(W2)

    st1 = pl.pallas_call(
        functools.partial(_grp_p1, G),
        grid=(grid,),
        in_specs=[pl.BlockSpec((T, O), lambda i: (i, 0)),
                  pl.BlockSpec((G, O), lambda i: (i, 0))],
        out_specs=pl.BlockSpec((8, O), lambda i: (0, 0)),
        out_shape=jax.ShapeDtypeStruct((8, O), jnp.float32),
    )(u, cv)
    sc1, sh1 = _affine(st1, R, g1, b1)

    h2, st2 = pl.pallas_call(
        functools.partial(_grp_p2, G),
        grid=(grid,),
        in_specs=[pl.BlockSpec((T, O), lambda i: (i, 0)),
                  pl.BlockSpec((G, O), lambda i: (i, 0)),
                  pl.BlockSpec((1, O), lambda i: (0, 0)),
                  pl.BlockSpec((1, O), lambda i: (0, 0)),
                  pl.BlockSpec((O, O), lambda i: (0, 0))],
        out_specs=(pl.BlockSpec((T, O), lambda i: (i, 0)),
                   pl.BlockSpec((8, O), lambda i: (0, 0))),
        out_shape=(jax.ShapeDtypeStruct((R, O), jnp.float32),
                   jax.ShapeDtypeStruct((8, O), jnp.float32)),
    )(u, cv, sc1, sh1, w2T)
    sc2, sh2 = _affine(st2, R, g2, b2)

    out = pl.pallas_call(
        functools.partial(_grp_p3, G),
        grid=(grid,),
        in_specs=[pl.BlockSpec((T, O), lambda i: (i, 0)),
                  pl.BlockSpec((1, O), lambda i: (0, 0)),
                  pl.BlockSpec((1, O), lambda i: (0, 0))],
        out_specs=pl.BlockSpec((G, O), lambda i: (i, 0)),
        out_shape=jax.ShapeDtypeStruct((R // K, O), jnp.float32),
    )(h2, sc2, sh2)
    return out


def _tr_kernel(rows_ref, out_ref):
    S, O = rows_ref.shape[1], rows_ref.shape[2]
    ii = lax.broadcasted_iota(jnp.int32, (S, S), 0)
    jj = lax.broadcasted_iota(jnp.int32, (S, S), 1)
    eye = (ii == jj).astype(jnp.float32)
    out_ref[0] = lax.dot_general(rows_ref[0], eye, (((0,), (0,)), ((), ())),
                                 preferred_element_type=jnp.float32)


def kernel(x, w1, g1, be1, w2, g2, be2,
           s1w1, s1g1, s1be1, s1w2, s1g2, s1be2,
           s2w1, s2g1, s2be1, s2w2, s2g2, s2be2):
    S1, S2 = 512, 256

    xp8 = jnp.concatenate([x, jnp.zeros((B, 5, N), jnp.float32)], axis=1)
    xT = jnp.transpose(xp8, (0, 2, 1))
    w1Tp = jnp.concatenate(
        [jnp.transpose(w1), jnp.zeros((5, 64), jnp.float32)], axis=0)

    h1, st1 = pl.pallas_call(
        _init_p1,
        grid=(B,),
        in_specs=[pl.BlockSpec((1, N, 8), lambda i: (i, 0, 0)),
                  pl.BlockSpec((8, 64), lambda i: (0, 0))],
        out_specs=(pl.BlockSpec((1, N, 64), lambda i: (i, 0, 0)),
                   pl.BlockSpec((8, 64), lambda i: (0, 0))),
        out_shape=(jax.ShapeDtypeStruct((B, N, 64), jnp.float32),
                   jax.ShapeDtypeStruct((8, 64), jnp.float32)),
    )(xT, w1Tp)
    sc1, sh1 = _affine(st1, B * N, g1, be1)

    h2, st2 = pl.pallas_call(
        _init_p2,
        grid=(B,),
        in_specs=[pl.BlockSpec((1, N, 64), lambda i: (i, 0, 0)),
                  pl.BlockSpec((1, 64), lambda i: (0, 0)),
                  pl.BlockSpec((1, 64), lambda i: (0, 0)),
                  pl.BlockSpec((64, 64), lambda i: (0, 0))],
        out_specs=(pl.BlockSpec((1, N, 64), lambda i: (i, 0, 0)),
                   pl.BlockSpec((8, 64), lambda i: (0, 0))),
        out_shape=(jax.ShapeDtypeStruct((B, N, 64), jnp.float32),
                   jax.ShapeDtypeStruct((8, 64), jnp.float32)),
    )(h1, sc1, sh1, jnp.transpose(w2))
    sc2, sh2 = _affine(st2, B * N, g2, be2)

    feats = pl.pallas_call(
        _init_p3,
        grid=(B,),
        in_specs=[pl.BlockSpec((1, N, 64), lambda i: (i, 0, 0)),
                  pl.BlockSpec((1, 64), lambda i: (0, 0)),
                  pl.BlockSpec((1, 64), lambda i: (0, 0))],
        out_specs=pl.BlockSpec((1, N, 64), lambda i: (i, 0, 0)),
        out_shape=jax.ShapeDtypeStruct((B, N, 64), jnp.float32),
    )(h2, sc2, sh2)
    feats_flat = feats.reshape(B * N, 64)

    fps1, nxz1, nxT1 = _fps(x, S1, N)
    xyz = jnp.transpose(x[:, :3, :], (0, 2, 1))
    knn1 = _knn(nxT1, xyz, S1, N)

    u1t, v1t = _xform(feats_flat, s1w1)
    cv1 = _gather_rows(v1t, fps1.reshape(-1))
    u1 = _gather_rows(u1t, knn1.reshape(-1))
    feats1 = _group_stage(u1, cv1, s1w2, s1g1, s1be1,
                          s1g2, s1be2)

    fps2, _, nxT2 = _fps(nxz1, S2, S1)
    knn2 = _knn(nxT2, nxT1, S2, S1)

    u2t, v2t = _xform(feats1, s2w1)
    cv2 = _gather_rows(v2t, fps2.reshape(-1))
    u2 = _gather_rows(u2t, knn2.reshape(-1))
    feats2 = _group_stage(u2, cv2, s2w2, s2g1, s2be1,
                          s2g2, s2be2)

    out = pl.pallas_call(
        _tr_kernel,
        grid=(B,),
        in_specs=[pl.BlockSpec((1, S2, 256), lambda i: (i, 0, 0))],
        out_specs=pl.BlockSpec((1, 256, S2), lambda i: (i, 0, 0)),
        out_shape=jax.ShapeDtypeStruct((B, 256, S2), jnp.float32),
    )(feats2.reshape(B, S2, 256))
    return out
